# Initial kernel scaffold; baseline (speedup 1.0000x reference)
import jax
import jax.numpy as jnp
from jax.experimental import pallas as pl


def kernel(query_type, node_operator, Q_table, N_table, W1, b1, W2, b2):
    raise NotImplementedError("write your pallas kernel here")


# speedup vs baseline: 4.4059x; 4.4059x over previous
import functools

import jax
import jax.numpy as jnp
from jax import lax
from jax.experimental import pallas as pl
from jax.experimental.pallas import tpu as pltpu
from jax.experimental.pallas import tpu_sc as plsc

B = 4096
N = 50
D = 64
NQ = 4
NN = 20
NF = NQ * NN
BN = B * N

NC = 2
NS = 16
NW = NC * NS
ROWS_PER_W = BN // NW
CHUNK = 128
NCHUNK = ROWS_PER_W // CHUNK


def _table_body(q_ref, n_ref, w1_ref, b1_ref, w2_ref, b2_ref, f_ref):
    rq = lax.broadcasted_iota(jnp.int32, (NF, NQ), 0) // NN
    cq = lax.broadcasted_iota(jnp.int32, (NF, NQ), 1)
    oh_q = jnp.where(rq == cq, 1.0, 0.0).astype(jnp.float32)
    rn = lax.broadcasted_iota(jnp.int32, (NF, NN), 0) % NN
    cn = lax.broadcasted_iota(jnp.int32, (NF, NN), 1)
    oh_n = jnp.where(rn == cn, 1.0, 0.0).astype(jnp.float32)
    qe = jnp.dot(oh_q, q_ref[...], preferred_element_type=jnp.float32)
    ne = jnp.dot(oh_n, n_ref[...], preferred_element_type=jnp.float32)
    combined = jnp.concatenate([qe, ne * qe], axis=-1)
    h = jnp.maximum(
        jnp.dot(combined, w1_ref[...], preferred_element_type=jnp.float32)
        + b1_ref[...],
        0.0,
    )
    res = jnp.dot(h, w2_ref[...], preferred_element_type=jnp.float32) + b2_ref[...]
    f_ref[...] = jnp.concatenate(
        [res, jnp.zeros((NF, 128 - D), jnp.float32)], axis=-1
    )


_table_call = pl.pallas_call(
    _table_body,
    out_shape=jax.ShapeDtypeStruct((NF, 128), jnp.float32),
)


@functools.cache
def _build_sc_gather():
    @functools.partial(
        pl.kernel,
        mesh=plsc.VectorSubcoreMesh(core_axis_name="c", subcore_axis_name="s"),
        out_type=jax.ShapeDtypeStruct((BN, 128), jnp.float32),
        scratch_types=[
            pltpu.VMEM((NCHUNK, CHUNK), jnp.int32),
            pltpu.VMEM((NCHUNK, CHUNK), jnp.int32),
            pltpu.VMEM((NCHUNK, CHUNK), jnp.int32),
            pltpu.VMEM((CHUNK, 128), jnp.float32),
            pltpu.SemaphoreType.DMA,
        ],
    )
    def _sc_gather(f_hbm, q_hbm, n_hbm, out_hbm, q_v, n_v, idx_v, rows_v, sem):
        wid = lax.axis_index("s") * NC + lax.axis_index("c")
        pltpu.sync_copy(q_hbm.at[wid], q_v)
        pltpu.sync_copy(n_hbm.at[wid], n_v)

        def fuse_body(j, carry):
            for c in range(CHUNK // 16):
                s = pl.ds(c * 16, 16)
                idx_v[j, s] = q_v[j, s] * NN + n_v[j, s]
            return carry

        lax.fori_loop(0, NCHUNK, fuse_body, 0)

        base = wid * ROWS_PER_W

        def gs_body(j, carry):
            pltpu.async_copy(f_hbm.at[idx_v.at[j]], rows_v, sem).wait()
            pltpu.sync_copy(rows_v, out_hbm.at[pl.ds(base + j * CHUNK, CHUNK)])
            return carry

        lax.fori_loop(0, NCHUNK, gs_body, 0)

    return _sc_gather


def kernel(query_type, node_operator, Q_table, N_table, W1, b1, W2, b2):
    fused_table = _table_call(
        Q_table, N_table, W1, b1.reshape(1, D), W2, b2.reshape(1, D)
    )
    if query_type.dtype != jnp.int32:
        query_type = query_type.astype(jnp.int32)
    if node_operator.dtype != jnp.int32:
        node_operator = node_operator.astype(jnp.int32)
    q3 = query_type.reshape(NW, NCHUNK, CHUNK)
    n3 = node_operator.reshape(NW, NCHUNK, CHUNK)
    out = _build_sc_gather()(fused_table, q3, n3)
    return out[:, :D].reshape(B, N, D)


# speedup vs baseline: 4.4355x; 1.0067x over previous
import functools

import jax
import jax.numpy as jnp
from jax import lax
from jax.experimental import pallas as pl
from jax.experimental.pallas import tpu as pltpu
from jax.experimental.pallas import tpu_sc as plsc

B = 4096
N = 50
D = 64
NQ = 4
NN = 20
NF = NQ * NN
BN = B * N

NC = 2
NS = 16
NW = NC * NS
ROWS_PER_W = BN // NW
CHUNK = 128
NCHUNK = ROWS_PER_W // CHUNK


def _table_body(q_ref, n_ref, w1_ref, b1_ref, w2_ref, b2_ref, f_ref):
    rq = lax.broadcasted_iota(jnp.int32, (NF, NQ), 0) // NN
    cq = lax.broadcasted_iota(jnp.int32, (NF, NQ), 1)
    oh_q = jnp.where(rq == cq, 1.0, 0.0).astype(jnp.float32)
    rn = lax.broadcasted_iota(jnp.int32, (NF, NN), 0) % NN
    cn = lax.broadcasted_iota(jnp.int32, (NF, NN), 1)
    oh_n = jnp.where(rn == cn, 1.0, 0.0).astype(jnp.float32)
    qe = jnp.dot(oh_q, q_ref[...], preferred_element_type=jnp.float32)
    ne = jnp.dot(oh_n, n_ref[...], preferred_element_type=jnp.float32)
    combined = jnp.concatenate([qe, ne * qe], axis=-1)
    h = jnp.maximum(
        jnp.dot(combined, w1_ref[...], preferred_element_type=jnp.float32)
        + b1_ref[...],
        0.0,
    )
    res = jnp.dot(h, w2_ref[...], preferred_element_type=jnp.float32) + b2_ref[...]
    f_ref[...] = jnp.concatenate(
        [res, jnp.zeros((NF, 128 - D), jnp.float32)], axis=-1
    )


_table_call = pl.pallas_call(
    _table_body,
    out_shape=jax.ShapeDtypeStruct((NF, 128), jnp.float32),
)


NBUF = 5


@functools.cache
def _build_sc_gather():
    @functools.partial(
        pl.kernel,
        mesh=plsc.VectorSubcoreMesh(core_axis_name="c", subcore_axis_name="s"),
        out_type=jax.ShapeDtypeStruct((BN, 128), jnp.float32),
        scratch_types=[
            pltpu.VMEM((NCHUNK, CHUNK), jnp.int32),
            pltpu.VMEM((NCHUNK, CHUNK), jnp.int32),
            pltpu.VMEM((NCHUNK, CHUNK), jnp.int32),
            pltpu.VMEM((NBUF, CHUNK, 128), jnp.float32),
            pltpu.SemaphoreType.DMA((NBUF,)),
            pltpu.SemaphoreType.DMA((NBUF,)),
        ],
    )
    def _sc_gather(f_hbm, q_hbm, n_hbm, out_hbm, q_v, n_v, idx_v, rows_v,
                   gsem, ssem):
        wid = lax.axis_index("s") * NC + lax.axis_index("c")
        pltpu.sync_copy(q_hbm.at[wid], q_v)
        pltpu.sync_copy(n_hbm.at[wid], n_v)

        def fuse_body(j, carry):
            for c in range(CHUNK // 16):
                s = pl.ds(c * 16, 16)
                idx_v[j, s] = q_v[j, s] * NN + n_v[j, s]
            return carry

        lax.fori_loop(0, NCHUNK, fuse_body, 0)

        base = wid * ROWS_PER_W

        def gather(j, b):
            pltpu.async_copy(f_hbm.at[idx_v.at[j]], rows_v.at[b], gsem.at[b])

        def scatter(j, b):
            pltpu.async_copy(
                rows_v.at[b],
                out_hbm.at[pl.ds(base + j * CHUNK, CHUNK)],
                ssem.at[b],
            )

        for b in range(NBUF - 1):
            gather(b, b)

        for j in range(NCHUNK):
            b = j % NBUF
            pltpu.make_async_copy(f_hbm.at[idx_v.at[j]], rows_v.at[b],
                                  gsem.at[b]).wait()
            scatter(j, b)
            jn = j + NBUF - 1
            if jn < NCHUNK:
                bn = jn % NBUF
                if j >= 1:
                    pltpu.make_async_copy(
                        rows_v.at[bn],
                        out_hbm.at[pl.ds(base, CHUNK)],
                        ssem.at[bn],
                    ).wait()
                gather(jn, bn)

        for b in range(NBUF):
            pltpu.make_async_copy(
                rows_v.at[b],
                out_hbm.at[pl.ds(base, CHUNK)],
                ssem.at[b],
            ).wait()

    return _sc_gather


def kernel(query_type, node_operator, Q_table, N_table, W1, b1, W2, b2):
    fused_table = _table_call(
        Q_table, N_table, W1, b1.reshape(1, D), W2, b2.reshape(1, D)
    )
    if query_type.dtype != jnp.int32:
        query_type = query_type.astype(jnp.int32)
    if node_operator.dtype != jnp.int32:
        node_operator = node_operator.astype(jnp.int32)
    q3 = query_type.reshape(NW, NCHUNK, CHUNK)
    n3 = node_operator.reshape(NW, NCHUNK, CHUNK)
    out = _build_sc_gather()(fused_table, q3, n3)
    return out[:, :D].reshape(B, N, D)


# speedup vs baseline: 8.2847x; 1.8678x over previous
import functools

import jax
import jax.numpy as jnp
from jax import lax
from jax.experimental import pallas as pl
from jax.experimental.pallas import tpu as pltpu
from jax.experimental.pallas import tpu_sc as plsc

B = 4096
N = 50
D = 64
NQ = 4
NN = 20
NF = NQ * NN
BN = B * N

NC = 2
NS = 16
NW = NC * NS
ROWS_PER_W = BN // NW
CHUNK = 128
NCHUNK = ROWS_PER_W // CHUNK


def _table_body(q_ref, n_ref, w1_ref, b1_ref, w2_ref, b2_ref, f_ref):
    rq = lax.broadcasted_iota(jnp.int32, (NF, NQ), 0) // NN
    cq = lax.broadcasted_iota(jnp.int32, (NF, NQ), 1)
    oh_q = jnp.where(rq == cq, 1.0, 0.0).astype(jnp.float32)
    rn = lax.broadcasted_iota(jnp.int32, (NF, NN), 0) % NN
    cn = lax.broadcasted_iota(jnp.int32, (NF, NN), 1)
    oh_n = jnp.where(rn == cn, 1.0, 0.0).astype(jnp.float32)
    qe = jnp.dot(oh_q, q_ref[...], preferred_element_type=jnp.float32)
    ne = jnp.dot(oh_n, n_ref[...], preferred_element_type=jnp.float32)
    combined = jnp.concatenate([qe, ne * qe], axis=-1)
    h = jnp.maximum(
        jnp.dot(combined, w1_ref[...], preferred_element_type=jnp.float32)
        + b1_ref[...],
        0.0,
    )
    res = jnp.dot(h, w2_ref[...], preferred_element_type=jnp.float32) + b2_ref[...]
    f_ref[...] = jnp.concatenate(
        [res, jnp.zeros((NF, 128 - D), jnp.float32)], axis=-1
    )


_table_call = pl.pallas_call(
    _table_body,
    out_shape=jax.ShapeDtypeStruct((NF, 128), jnp.float32),
)


NBUF = 5


@functools.cache
def _build_sc_gather():
    @functools.partial(
        pl.kernel,
        mesh=plsc.VectorSubcoreMesh(core_axis_name="c", subcore_axis_name="s"),
        out_type=jax.ShapeDtypeStruct((BN, 128), jnp.float32),
        scratch_types=[
            pltpu.VMEM((NCHUNK, CHUNK), jnp.int32),
            pltpu.VMEM((NCHUNK, CHUNK), jnp.int32),
            pltpu.VMEM((NCHUNK, CHUNK), jnp.int32),
            pltpu.VMEM((NBUF, CHUNK, 128), jnp.float32),
            pltpu.SemaphoreType.DMA((NBUF,)),
            pltpu.SemaphoreType.DMA((NBUF,)),
        ],
    )
    def _sc_gather(f_hbm, q_hbm, n_hbm, out_hbm, q_v, n_v, idx_v, rows_v,
                   gsem, ssem):
        wid = lax.axis_index("s") * NC + lax.axis_index("c")
        pltpu.sync_copy(q_hbm.at[wid], q_v)
        pltpu.sync_copy(n_hbm.at[wid], n_v)

        tbase = wid * NF

        def fuse_body(j, carry):
            for c in range(CHUNK // 16):
                s = pl.ds(c * 16, 16)
                idx_v[j, s] = q_v[j, s] * NN + n_v[j, s] + tbase
            return carry

        lax.fori_loop(0, NCHUNK, fuse_body, 0)

        base = wid * ROWS_PER_W

        def gather(j, b):
            pltpu.async_copy(f_hbm.at[idx_v.at[j]], rows_v.at[b], gsem.at[b])

        def scatter(j, b):
            pltpu.async_copy(
                rows_v.at[b],
                out_hbm.at[pl.ds(base + j * CHUNK, CHUNK)],
                ssem.at[b],
            )

        for b in range(NBUF - 1):
            gather(b, b)

        for j in range(NCHUNK):
            b = j % NBUF
            pltpu.make_async_copy(f_hbm.at[idx_v.at[j]], rows_v.at[b],
                                  gsem.at[b]).wait()
            scatter(j, b)
            jn = j + NBUF - 1
            if jn < NCHUNK:
                bn = jn % NBUF
                if j >= 1:
                    pltpu.make_async_copy(
                        rows_v.at[bn],
                        out_hbm.at[pl.ds(base, CHUNK)],
                        ssem.at[bn],
                    ).wait()
                gather(jn, bn)

        for b in range(NBUF):
            pltpu.make_async_copy(
                rows_v.at[b],
                out_hbm.at[pl.ds(base, CHUNK)],
                ssem.at[b],
            ).wait()

    return _sc_gather


def kernel(query_type, node_operator, Q_table, N_table, W1, b1, W2, b2):
    fused_table = _table_call(
        Q_table, N_table, W1, b1.reshape(1, D), W2, b2.reshape(1, D)
    )
    fused_table = jnp.broadcast_to(fused_table[None], (NW, NF, 128)).reshape(
        NW * NF, 128
    )
    if query_type.dtype != jnp.int32:
        query_type = query_type.astype(jnp.int32)
    if node_operator.dtype != jnp.int32:
        node_operator = node_operator.astype(jnp.int32)
    q3 = query_type.reshape(NW, NCHUNK, CHUNK)
    n3 = node_operator.reshape(NW, NCHUNK, CHUNK)
    out = _build_sc_gather()(fused_table, q3, n3)
    return out[:, :D].reshape(B, N, D)


# speedup vs baseline: 9.7257x; 1.1739x over previous
import functools

import jax
import jax.numpy as jnp
from jax import lax
from jax.experimental import pallas as pl
from jax.experimental.pallas import tpu as pltpu
from jax.experimental.pallas import tpu_sc as plsc

B = 4096
N = 50
D = 64
NQ = 4
NN = 20
NF = NQ * NN
BN = B * N

NC = 2
NS = 16
NW = NC * NS
BATCH_PER_W = B // NW
NPAD = 64
NROW = 56
NBUF = 4


def _table_body(q_ref, n_ref, w1_ref, b1_ref, w2_ref, b2_ref, f_ref):
    rq = lax.broadcasted_iota(jnp.int32, (NF, NQ), 0) // NN
    cq = lax.broadcasted_iota(jnp.int32, (NF, NQ), 1)
    oh_q = jnp.where(rq == cq, 1.0, 0.0).astype(jnp.float32)
    rn = lax.broadcasted_iota(jnp.int32, (NF, NN), 0) % NN
    cn = lax.broadcasted_iota(jnp.int32, (NF, NN), 1)
    oh_n = jnp.where(rn == cn, 1.0, 0.0).astype(jnp.float32)
    qe = jnp.dot(oh_q, q_ref[...], preferred_element_type=jnp.float32)
    ne = jnp.dot(oh_n, n_ref[...], preferred_element_type=jnp.float32)
    combined = jnp.concatenate([qe, ne * qe], axis=-1)
    h = jnp.maximum(
        jnp.dot(combined, w1_ref[...], preferred_element_type=jnp.float32)
        + b1_ref[...],
        0.0,
    )
    res = jnp.dot(h, w2_ref[...], preferred_element_type=jnp.float32) + b2_ref[...]
    f_ref[...] = jnp.concatenate(
        [res, jnp.zeros((NF, 128 - D), jnp.float32)], axis=-1
    )


_table_call = pl.pallas_call(
    _table_body,
    out_shape=jax.ShapeDtypeStruct((NF, 128), jnp.float32),
)


@functools.cache
def _build_sc_gather():
    @functools.partial(
        pl.kernel,
        mesh=plsc.VectorSubcoreMesh(core_axis_name="c", subcore_axis_name="s"),
        out_type=jax.ShapeDtypeStruct((B, NROW, 128), jnp.float32),
        scratch_types=[
            pltpu.VMEM((BATCH_PER_W, NPAD), jnp.int32),
            pltpu.VMEM((BATCH_PER_W, NPAD), jnp.int32),
            pltpu.VMEM((BATCH_PER_W, NPAD), jnp.int32),
            pltpu.VMEM((NBUF, NROW, 128), jnp.float32),
            pltpu.SemaphoreType.DMA((NBUF,)),
            pltpu.SemaphoreType.DMA((NBUF,)),
        ],
    )
    def _sc_gather(f_hbm, q_hbm, n_hbm, out_hbm, q_v, n_v, idx_v, rows_v,
                   gsem, ssem):
        wid = lax.axis_index("s") * NC + lax.axis_index("c")
        pltpu.sync_copy(q_hbm.at[wid], q_v)
        pltpu.sync_copy(n_hbm.at[wid], n_v)

        tbase = wid * NF

        def fuse_body(j, carry):
            for c in range(NPAD // 16):
                s = pl.ds(c * 16, 16)
                idx_v[j, s] = q_v[j, s] * NN + n_v[j, s] + tbase
            return carry

        lax.fori_loop(0, BATCH_PER_W, fuse_body, 0)

        bbase = wid * BATCH_PER_W

        def gather(j, b):
            pltpu.async_copy(
                f_hbm.at[idx_v.at[j, pl.ds(0, NROW)]],
                rows_v.at[b],
                gsem.at[b],
            )

        def wait_gather(b):
            pltpu.make_async_copy(
                f_hbm.at[idx_v.at[0, pl.ds(0, NROW)]],
                rows_v.at[b],
                gsem.at[b],
            ).wait()

        def scatter(j, b):
            pltpu.async_copy(rows_v.at[b], out_hbm.at[bbase + j], ssem.at[b])

        def wait_scatter(b):
            pltpu.make_async_copy(
                rows_v.at[b], out_hbm.at[bbase], ssem.at[b]
            ).wait()

        for b in range(NBUF - 1):
            gather(b, b)

        def step(g, carry):
            for b0 in range(NBUF):
                j = g * NBUF + b0
                wait_gather(b0)
                scatter(j, b0)
                jn = j + NBUF - 1
                bn = (b0 + NBUF - 1) % NBUF

                @pl.when(jn < BATCH_PER_W)
                def _():
                    @pl.when(j >= 1)
                    def _():
                        wait_scatter(bn)

                    gather(jn, bn)

            return carry

        lax.fori_loop(0, BATCH_PER_W // NBUF, step, 0)

        for b in range(NBUF):
            wait_scatter(b)

    return _sc_gather


def kernel(query_type, node_operator, Q_table, N_table, W1, b1, W2, b2):
    fused_table = _table_call(
        Q_table, N_table, W1, b1.reshape(1, D), W2, b2.reshape(1, D)
    )
    fused_table = jnp.broadcast_to(fused_table[None], (NW, NF, 128)).reshape(
        NW * NF, 128
    )
    if query_type.dtype != jnp.int32:
        query_type = query_type.astype(jnp.int32)
    if node_operator.dtype != jnp.int32:
        node_operator = node_operator.astype(jnp.int32)
    qp = jnp.pad(query_type, ((0, 0), (0, NPAD - N))).reshape(
        NW, BATCH_PER_W, NPAD
    )
    np_ = jnp.pad(node_operator, ((0, 0), (0, NPAD - N))).reshape(
        NW, BATCH_PER_W, NPAD
    )
    out = _build_sc_gather()(fused_table, qp, np_)
    return out[:, :N, :D]
